# baseline (speedup 1.0000x reference)
import functools
import math

import jax
import jax.numpy as jnp
from jax import lax
from jax.experimental import pallas as pl
from jax.experimental.pallas import tpu as pltpu

_NEG = -1e30
_BR = 1024
_BC = 2048


def _kv_proj_kernel(x_ref, bid_ref, mk_ref, wk_ref, bk_ref, wv_ref, bv_ref,
                    kt_ref, v_ref, *, kq):
    x = x_ref[...]
    kt = lax.dot_general(wk_ref[...], x, (((1,), (1,)), ((), ())),
                         preferred_element_type=jnp.float32)
    kt_ref[0:kq, :] = kt + bk_ref[...]
    kt_ref[kq:kq + 1, :] = jnp.where(mk_ref[0] > 0.5,
                                     bid_ref[0].astype(jnp.float32), 127.0)
    v_ref[...] = lax.dot_general(x, wv_ref[...], (((1,), (1,)), ((), ())),
                                 preferred_element_type=jnp.float32) + bv_ref[...]


def _attn_kernel(sinfo_ref, x_ref, bid_ref, mk_ref, wq_ref, bq_ref,
                 kt_hbm, v_hbm, out_ref, kt_buf, v_buf, sem, *, kq, c_out):
    i = pl.program_id(0)
    cb0 = sinfo_ref[0, i]
    ncb = sinfo_ref[1, i]
    x = x_ref[...]
    q = lax.dot_general(x, wq_ref[...], (((1,), (1,)), ((), ())),
                        preferred_element_type=jnp.float32)
    q = (q + bq_ref[...]) * (1.0 / math.sqrt(kq))
    bid_row = bid_ref[0].astype(jnp.float32)

    def _copies(t, slot):
        c = cb0 + t
        return (
            pltpu.make_async_copy(
                kt_hbm.at[:, pl.ds(c * _BC, _BC)], kt_buf.at[slot],
                sem.at[0, slot]),
            pltpu.make_async_copy(
                v_hbm.at[pl.ds(c * _BC, _BC), :], v_buf.at[slot],
                sem.at[1, slot]),
        )

    for cp in _copies(0, 0):
        cp.start()

    def body(t, carry):
        m, l, acc = carry
        slot = lax.rem(t, 2)

        @pl.when(t + 1 < ncb)
        def _prefetch():
            for cp in _copies(t + 1, lax.rem(t + 1, 2)):
                cp.start()

        for cp in _copies(t, slot):
            cp.wait()

        kt = kt_buf[slot, 0:kq, :]
        bid_col = kt_buf[slot, kq:kq + 1, :]
        s = jnp.dot(q, kt, preferred_element_type=jnp.float32)
        s = jnp.where(bid_row == bid_col, s, _NEG)
        m_new = jnp.maximum(m, jnp.max(s, axis=1, keepdims=True))
        alpha = jnp.exp(m - m_new)
        p = jnp.exp(s - m_new)
        l_new = l * alpha + jnp.sum(p, axis=1, keepdims=True)
        acc_new = acc * alpha + jnp.dot(p, v_buf[slot],
                                        preferred_element_type=jnp.float32)
        return m_new, l_new, acc_new

    m0 = jnp.full((_BR, 1), _NEG, jnp.float32)
    l0 = jnp.zeros((_BR, 1), jnp.float32)
    acc0 = jnp.zeros((_BR, c_out), jnp.float32)
    m, l, acc = lax.fori_loop(0, ncb, body, (m0, l0, acc0))

    keep_row = mk_ref[0] > 0.5
    replace = jnp.logical_and(jnp.logical_not(keep_row), m > _NEG * 0.5)
    out_ref[...] = jnp.where(replace, acc / l, x)


def kernel(features, mask_keep, Wq, bq, Wk, bk, Wv, bv, batch_ids):
    n, c_in = features.shape
    kq = Wq.shape[0]
    c_out = Wv.shape[0]
    kt_rows = ((kq + 2 + 7) // 8) * 8
    nbr = n // _BR
    nbc = n // _BC

    bid32 = batch_ids.astype(jnp.int32)
    b_lo = bid32[::_BR]
    b_hi = bid32[_BR - 1::_BR]
    col_lo = jnp.searchsorted(bid32, b_lo, side="left")
    col_hi = jnp.searchsorted(bid32, b_hi, side="right")
    cb_lo = (col_lo // _BC).astype(jnp.int32)
    cb_hi = ((col_hi - 1) // _BC).astype(jnp.int32)
    sinfo = jnp.stack([cb_lo, cb_hi - cb_lo + 1])

    bid_cols = bid32.reshape(nbc, 1, _BC)
    mk_cols = mask_keep.reshape(nbc, 1, _BC)
    bid_rows = bid32.reshape(nbr, _BR, 1)
    mk_rows = mask_keep.reshape(nbr, _BR, 1)
    bk2 = bk.reshape(kq, 1)
    bq2 = bq.reshape(1, kq)
    bv2 = bv.reshape(1, c_out)

    kt_ext, v = pl.pallas_call(
        functools.partial(_kv_proj_kernel, kq=kq),
        grid=(nbc,),
        in_specs=[
            pl.BlockSpec((_BC, c_in), lambda j: (j, 0)),
            pl.BlockSpec((1, 1, _BC), lambda j: (j, 0, 0)),
            pl.BlockSpec((1, 1, _BC), lambda j: (j, 0, 0)),
            pl.BlockSpec((kq, c_in), lambda j: (0, 0)),
            pl.BlockSpec((kq, 1), lambda j: (0, 0)),
            pl.BlockSpec((c_out, c_in), lambda j: (0, 0)),
            pl.BlockSpec((1, c_out), lambda j: (0, 0)),
        ],
        out_specs=[
            pl.BlockSpec((kt_rows, _BC), lambda j: (0, j)),
            pl.BlockSpec((_BC, c_out), lambda j: (j, 0)),
        ],
        out_shape=[
            jax.ShapeDtypeStruct((kt_rows, n), jnp.float32),
            jax.ShapeDtypeStruct((n, c_out), jnp.float32),
        ],
        compiler_params=pltpu.CompilerParams(
            dimension_semantics=("parallel",)),
    )(features, bid_cols, mk_cols.reshape(nbc, 1, _BC), Wk, bk2, Wv, bv2)

    out = pl.pallas_call(
        functools.partial(_attn_kernel, kq=kq, c_out=c_out),
        grid_spec=pltpu.PrefetchScalarGridSpec(
            num_scalar_prefetch=1,
            grid=(nbr,),
            in_specs=[
                pl.BlockSpec((_BR, c_in), lambda i, s: (i, 0)),
                pl.BlockSpec((1, _BR, 1), lambda i, s: (i, 0, 0)),
                pl.BlockSpec((1, _BR, 1), lambda i, s: (i, 0, 0)),
                pl.BlockSpec((kq, c_in), lambda i, s: (0, 0)),
                pl.BlockSpec((1, kq), lambda i, s: (0, 0)),
                pl.BlockSpec(memory_space=pl.ANY),
                pl.BlockSpec(memory_space=pl.ANY),
            ],
            out_specs=pl.BlockSpec((_BR, c_in), lambda i, s: (i, 0)),
            scratch_shapes=[
                pltpu.VMEM((2, kt_rows, _BC), jnp.float32),
                pltpu.VMEM((2, _BC, c_out), jnp.float32),
                pltpu.SemaphoreType.DMA((2, 2)),
            ],
        ),
        out_shape=jax.ShapeDtypeStruct((n, c_in), jnp.float32),
        compiler_params=pltpu.CompilerParams(
            dimension_semantics=("arbitrary",)),
    )(sinfo, features, bid_rows, mk_rows, Wq, bq2, kt_ext, v)

    return out


# speedup vs baseline: 1.0834x; 1.0834x over previous
import functools
import math

import jax
import jax.numpy as jnp
from jax import lax
from jax.experimental import pallas as pl
from jax.experimental.pallas import tpu as pltpu

_NEG = -1e30
_BR = 1024
_BC = 1024


def _kv_proj_kernel(x_ref, bid_ref, mk_ref, wk_ref, bk_ref, wv_ref, bv_ref,
                    kt_ref, v_ref, *, kq):
    x = x_ref[...]
    kt = lax.dot_general(wk_ref[...], x, (((1,), (1,)), ((), ())),
                         preferred_element_type=jnp.float32)
    kt_ref[0:kq, :] = (kt + bk_ref[...]).astype(jnp.bfloat16)
    kt_ref[kq:kq + 1, :] = jnp.where(mk_ref[0] > 0.5,
                                     bid_ref[0].astype(jnp.float32),
                                     127.0).astype(jnp.bfloat16)
    v_ref[...] = (lax.dot_general(x, wv_ref[...], (((1,), (1,)), ((), ())),
                                  preferred_element_type=jnp.float32)
                  + bv_ref[...]).astype(jnp.bfloat16)


def _attn_kernel(sinfo_ref, x_ref, bid_ref, mk_ref, wq_ref, bq_ref,
                 kt_hbm, v_hbm, out_ref, kt_buf, v_buf, sem, *, kq, c_out):
    i = pl.program_id(0)
    cb0 = sinfo_ref[0, i]
    ncb = sinfo_ref[1, i]
    x = x_ref[...]
    q = lax.dot_general(x, wq_ref[...], (((1,), (1,)), ((), ())),
                        preferred_element_type=jnp.float32)
    q = ((q + bq_ref[...]) * (1.0 / math.sqrt(kq))).astype(jnp.bfloat16)
    bid_row = bid_ref[0].astype(jnp.bfloat16)

    def _copies(t, slot):
        c = cb0 + t
        return (
            pltpu.make_async_copy(
                kt_hbm.at[:, pl.ds(c * _BC, _BC)], kt_buf.at[slot],
                sem.at[0, slot]),
            pltpu.make_async_copy(
                v_hbm.at[pl.ds(c * _BC, _BC), :], v_buf.at[slot],
                sem.at[1, slot]),
        )

    for cp in _copies(0, 0):
        cp.start()

    def body(t, carry):
        m, l, acc = carry
        slot = lax.rem(t, 2)

        @pl.when(t + 1 < ncb)
        def _prefetch():
            for cp in _copies(t + 1, lax.rem(t + 1, 2)):
                cp.start()

        for cp in _copies(t, slot):
            cp.wait()

        kt = kt_buf[slot, 0:kq, :]
        bid_col = kt_buf[slot, kq:kq + 1, :]
        s = jnp.dot(q, kt, preferred_element_type=jnp.float32)
        s = jnp.where(bid_row == bid_col, s, _NEG)
        m_new = jnp.maximum(m, jnp.max(s, axis=1, keepdims=True))
        alpha = jnp.exp(m - m_new)
        p = jnp.exp(s - m_new)
        l_new = l * alpha + jnp.sum(p, axis=1, keepdims=True)
        acc_new = acc * alpha + jnp.dot(p.astype(jnp.bfloat16), v_buf[slot],
                                        preferred_element_type=jnp.float32)
        return m_new, l_new, acc_new

    m0 = jnp.full((_BR, 1), _NEG, jnp.float32)
    l0 = jnp.zeros((_BR, 1), jnp.float32)
    acc0 = jnp.zeros((_BR, c_out), jnp.float32)
    m, l, acc = lax.fori_loop(0, ncb, body, (m0, l0, acc0))

    keep_row = mk_ref[0] > 0.5
    replace = jnp.logical_and(jnp.logical_not(keep_row), m > _NEG * 0.5)
    out_ref[...] = jnp.where(replace, acc / l, x)


def kernel(features, mask_keep, Wq, bq, Wk, bk, Wv, bv, batch_ids):
    n, c_in = features.shape
    kq = Wq.shape[0]
    c_out = Wv.shape[0]
    kt_rows = ((kq + 2 + 7) // 8) * 8
    nbr = n // _BR
    nbc = n // _BC

    bid32 = batch_ids.astype(jnp.int32)
    b_lo = bid32[::_BR]
    b_hi = bid32[_BR - 1::_BR]
    col_lo = jnp.searchsorted(bid32, b_lo, side="left")
    col_hi = jnp.searchsorted(bid32, b_hi, side="right")
    cb_lo = (col_lo // _BC).astype(jnp.int32)
    cb_hi = ((col_hi - 1) // _BC).astype(jnp.int32)
    sinfo = jnp.stack([cb_lo, cb_hi - cb_lo + 1])

    bid_cols = bid32.reshape(nbc, 1, _BC)
    mk_cols = mask_keep.reshape(nbc, 1, _BC)
    bid_rows = bid32.reshape(nbr, _BR, 1)
    mk_rows = mask_keep.reshape(nbr, _BR, 1)
    bk2 = bk.reshape(kq, 1)
    bq2 = bq.reshape(1, kq)
    bv2 = bv.reshape(1, c_out)

    kt_ext, v = pl.pallas_call(
        functools.partial(_kv_proj_kernel, kq=kq),
        grid=(nbc,),
        in_specs=[
            pl.BlockSpec((_BC, c_in), lambda j: (j, 0)),
            pl.BlockSpec((1, 1, _BC), lambda j: (j, 0, 0)),
            pl.BlockSpec((1, 1, _BC), lambda j: (j, 0, 0)),
            pl.BlockSpec((kq, c_in), lambda j: (0, 0)),
            pl.BlockSpec((kq, 1), lambda j: (0, 0)),
            pl.BlockSpec((c_out, c_in), lambda j: (0, 0)),
            pl.BlockSpec((1, c_out), lambda j: (0, 0)),
        ],
        out_specs=[
            pl.BlockSpec((kt_rows, _BC), lambda j: (0, j)),
            pl.BlockSpec((_BC, c_out), lambda j: (j, 0)),
        ],
        out_shape=[
            jax.ShapeDtypeStruct((kt_rows, n), jnp.bfloat16),
            jax.ShapeDtypeStruct((n, c_out), jnp.bfloat16),
        ],
        compiler_params=pltpu.CompilerParams(
            dimension_semantics=("parallel",)),
    )(features, bid_cols, mk_cols.reshape(nbc, 1, _BC), Wk, bk2, Wv, bv2)

    out = pl.pallas_call(
        functools.partial(_attn_kernel, kq=kq, c_out=c_out),
        grid_spec=pltpu.PrefetchScalarGridSpec(
            num_scalar_prefetch=1,
            grid=(nbr,),
            in_specs=[
                pl.BlockSpec((_BR, c_in), lambda i, s: (i, 0)),
                pl.BlockSpec((1, _BR, 1), lambda i, s: (i, 0, 0)),
                pl.BlockSpec((1, _BR, 1), lambda i, s: (i, 0, 0)),
                pl.BlockSpec((kq, c_in), lambda i, s: (0, 0)),
                pl.BlockSpec((1, kq), lambda i, s: (0, 0)),
                pl.BlockSpec(memory_space=pl.ANY),
                pl.BlockSpec(memory_space=pl.ANY),
            ],
            out_specs=pl.BlockSpec((_BR, c_in), lambda i, s: (i, 0)),
            scratch_shapes=[
                pltpu.VMEM((2, kt_rows, _BC), jnp.bfloat16),
                pltpu.VMEM((2, _BC, c_out), jnp.bfloat16),
                pltpu.SemaphoreType.DMA((2, 2)),
            ],
        ),
        out_shape=jax.ShapeDtypeStruct((n, c_in), jnp.float32),
        compiler_params=pltpu.CompilerParams(
            dimension_semantics=("arbitrary",)),
    )(sinfo, features, bid_rows, mk_rows, Wq, bq2, kt_ext, v)

    return out


# speedup vs baseline: 1.1289x; 1.0420x over previous
import functools
import math

import jax
import jax.numpy as jnp
from jax import lax
from jax.experimental import pallas as pl
from jax.experimental.pallas import tpu as pltpu

_NEG = -1e30
_BR = 1024
_BC = 1024


def _kv_proj_kernel(x_ref, bid_ref, mk_ref, wk_ref, bk_ref, wv_ref, bv_ref,
                    kt_ref, v_ref, *, kq):
    x = x_ref[...]
    kt = lax.dot_general(wk_ref[...], x, (((1,), (1,)), ((), ())),
                         preferred_element_type=jnp.float32)
    kt_ref[0:kq, :] = kt + bk_ref[...]
    kt_ref[kq:kq + 1, :] = jnp.where(mk_ref[0] > 0.5,
                                     bid_ref[0].astype(jnp.float32), 127.0)
    v_ref[...] = lax.dot_general(x, wv_ref[...], (((1,), (1,)), ((), ())),
                                 preferred_element_type=jnp.float32) + bv_ref[...]


def _attn_kernel(sinfo_ref, x_ref, bid_ref, mk_ref, wq_ref, bq_ref,
                 kt_hbm, v_hbm, out_ref, kt_buf, v_buf, sem, *, kq, c_out):
    i = pl.program_id(0)
    cb0 = sinfo_ref[0, i]
    ncb = sinfo_ref[1, i]
    x = x_ref[...]
    q = lax.dot_general(x, wq_ref[...], (((1,), (1,)), ((), ())),
                        preferred_element_type=jnp.float32)
    q = (q + bq_ref[...]) * (1.0 / math.sqrt(kq))
    bid_row = bid_ref[0].astype(jnp.float32)

    def _copies(t, slot):
        c = cb0 + t
        return (
            pltpu.make_async_copy(
                kt_hbm.at[:, pl.ds(c * _BC, _BC)], kt_buf.at[slot],
                sem.at[0, slot]),
            pltpu.make_async_copy(
                v_hbm.at[pl.ds(c * _BC, _BC), :], v_buf.at[slot],
                sem.at[1, slot]),
        )

    for cp in _copies(0, 0):
        cp.start()

    def body(t, carry):
        m, l, acc = carry
        slot = lax.rem(t, 2)

        @pl.when(t + 1 < ncb)
        def _prefetch():
            for cp in _copies(t + 1, lax.rem(t + 1, 2)):
                cp.start()

        for cp in _copies(t, slot):
            cp.wait()

        kt = kt_buf[slot, 0:kq, :]
        bid_col = kt_buf[slot, kq:kq + 1, :]
        s = jnp.dot(q, kt, preferred_element_type=jnp.float32)
        s = jnp.where(bid_row == bid_col, s, _NEG)
        m_new = jnp.maximum(m, jnp.max(s, axis=1, keepdims=True))
        alpha = jnp.exp(m - m_new)
        p = jnp.exp(s - m_new)
        l_new = l * alpha + jnp.sum(p, axis=1, keepdims=True)
        acc_new = acc * alpha + jnp.dot(p, v_buf[slot],
                                        preferred_element_type=jnp.float32)
        return m_new, l_new, acc_new

    m0 = jnp.full((_BR, 1), _NEG, jnp.float32)
    l0 = jnp.zeros((_BR, 1), jnp.float32)
    acc0 = jnp.zeros((_BR, c_out), jnp.float32)
    m, l, acc = lax.fori_loop(0, ncb, body, (m0, l0, acc0))

    keep_row = mk_ref[0] > 0.5
    replace = jnp.logical_and(jnp.logical_not(keep_row), m > _NEG * 0.5)
    out_ref[...] = jnp.where(replace, acc / l, x)


def kernel(features, mask_keep, Wq, bq, Wk, bk, Wv, bv, batch_ids):
    n, c_in = features.shape
    kq = Wq.shape[0]
    c_out = Wv.shape[0]
    kt_rows = ((kq + 2 + 7) // 8) * 8
    nbr = n // _BR
    nbc = n // _BC

    bid32 = batch_ids.astype(jnp.int32)
    b_lo = bid32[::_BR]
    b_hi = bid32[_BR - 1::_BR]
    col_lo = jnp.searchsorted(bid32, b_lo, side="left")
    col_hi = jnp.searchsorted(bid32, b_hi, side="right")
    cb_lo = (col_lo // _BC).astype(jnp.int32)
    cb_hi = ((col_hi - 1) // _BC).astype(jnp.int32)
    sinfo = jnp.stack([cb_lo, cb_hi - cb_lo + 1])

    bid_cols = bid32.reshape(nbc, 1, _BC)
    mk_cols = mask_keep.reshape(nbc, 1, _BC)
    bid_rows = bid32.reshape(nbr, _BR, 1)
    mk_rows = mask_keep.reshape(nbr, _BR, 1)
    bk2 = bk.reshape(kq, 1)
    bq2 = bq.reshape(1, kq)
    bv2 = bv.reshape(1, c_out)

    kt_ext, v = pl.pallas_call(
        functools.partial(_kv_proj_kernel, kq=kq),
        grid=(nbc,),
        in_specs=[
            pl.BlockSpec((_BC, c_in), lambda j: (j, 0)),
            pl.BlockSpec((1, 1, _BC), lambda j: (j, 0, 0)),
            pl.BlockSpec((1, 1, _BC), lambda j: (j, 0, 0)),
            pl.BlockSpec((kq, c_in), lambda j: (0, 0)),
            pl.BlockSpec((kq, 1), lambda j: (0, 0)),
            pl.BlockSpec((c_out, c_in), lambda j: (0, 0)),
            pl.BlockSpec((1, c_out), lambda j: (0, 0)),
        ],
        out_specs=[
            pl.BlockSpec((kt_rows, _BC), lambda j: (0, j)),
            pl.BlockSpec((_BC, c_out), lambda j: (j, 0)),
        ],
        out_shape=[
            jax.ShapeDtypeStruct((kt_rows, n), jnp.float32),
            jax.ShapeDtypeStruct((n, c_out), jnp.float32),
        ],
        compiler_params=pltpu.CompilerParams(
            dimension_semantics=("parallel",)),
    )(features, bid_cols, mk_cols.reshape(nbc, 1, _BC), Wk, bk2, Wv, bv2)

    out = pl.pallas_call(
        functools.partial(_attn_kernel, kq=kq, c_out=c_out),
        grid_spec=pltpu.PrefetchScalarGridSpec(
            num_scalar_prefetch=1,
            grid=(nbr,),
            in_specs=[
                pl.BlockSpec((_BR, c_in), lambda i, s: (i, 0)),
                pl.BlockSpec((1, _BR, 1), lambda i, s: (i, 0, 0)),
                pl.BlockSpec((1, _BR, 1), lambda i, s: (i, 0, 0)),
                pl.BlockSpec((kq, c_in), lambda i, s: (0, 0)),
                pl.BlockSpec((1, kq), lambda i, s: (0, 0)),
                pl.BlockSpec(memory_space=pl.ANY),
                pl.BlockSpec(memory_space=pl.ANY),
            ],
            out_specs=pl.BlockSpec((_BR, c_in), lambda i, s: (i, 0)),
            scratch_shapes=[
                pltpu.VMEM((2, kt_rows, _BC), jnp.float32),
                pltpu.VMEM((2, _BC, c_out), jnp.float32),
                pltpu.SemaphoreType.DMA((2, 2)),
            ],
        ),
        out_shape=jax.ShapeDtypeStruct((n, c_in), jnp.float32),
        compiler_params=pltpu.CompilerParams(
            dimension_semantics=("arbitrary",)),
    )(sinfo, features, bid_rows, mk_rows, Wq, bq2, kt_ext, v)

    return out
